# baseline (speedup 1.0000x reference)
import jax
import jax.numpy as jnp
from jax.experimental import pallas as pl
from jax.experimental.pallas import tpu as pltpu

N = 512
D = 128
H = 128
T = 16
B = 4
K = 3
C = 4
F32 = jnp.float32
BF16 = jnp.bfloat16


def _dot(a, b):
    return jnp.dot(a.astype(BF16), b.astype(BF16),
                   preferred_element_type=F32)


def _to_lane(v):
    return jnp.concatenate([v[bi * N:(bi + 1) * N] for bi in range(B)],
                           axis=1)


def _to_stacked(v, w):
    return jnp.concatenate([v[:, bi * w:(bi + 1) * w] for bi in range(B)],
                           axis=0)


def _layer(x0s, x1s, x2s, h0s, h0bs, h1s, h2s, sstack, w_in, bias, wg, wc):
    a = _dot(jnp.concatenate([x0s, x1s, x2s], axis=1), w_in) + bias
    g = jax.nn.sigmoid(
        a[:, :2 * H]
        + _dot(jnp.concatenate([h0bs, h1s, h2s], axis=1), wg))
    r, u = g[:, :H], g[:, H:]
    rs0 = (r * h0s).astype(BF16)
    rsd = _dot(sstack, _to_lane(rs0))
    rs1 = _to_stacked(rsd[:N], H).astype(BF16)
    rs2 = _to_stacked(rsd[N:], H).astype(BF16)
    c = jnp.tanh(
        a[:, 2 * H:]
        + _dot(jnp.concatenate([rs0, rs1, rs2], axis=1), wc))
    return u * h0s + (1.0 - u) * c


def _mega_body(idx_ref, x_ref, s_ref, w0_ref, b0_ref, wg0_ref, wc0_ref,
               w1_ref, b1_ref, wg1_ref, wc1_ref,
               wfc_ref, bfc_ref,
               o_ref, ss_ref, st0_ref, st1_ref, last_ref):
    t = pl.program_id(0)

    @pl.when(t == 0)
    def _():
        st0_ref[...] = jnp.zeros_like(st0_ref)
        st1_ref[...] = jnp.zeros_like(st1_ref)
        row = jax.lax.broadcasted_iota(jnp.int32, (N, N), 0)
        col = jax.lax.broadcasted_iota(jnp.int32, (N, N), 1)
        eye = (row == col).astype(F32)
        ss = s_ref[...]
        ss_ref[:N] = ss
        ss_ref[N:] = (2.0 * _dot(ss, ss) - eye).astype(BF16)

    sstack = ss_ref[...]

    h0s_l0 = st0_ref[...]
    h0s_l1 = st1_ref[...]
    h0bs_l0 = h0s_l0.astype(BF16)
    h0bs_l1 = h0s_l1.astype(BF16)

    x0s = jnp.concatenate([x_ref[bi, 0].astype(BF16) for bi in range(B)],
                          axis=0)

    wide = jnp.concatenate(
        [_to_lane(x0s), _to_lane(h0bs_l0), _to_lane(h0bs_l1)], axis=1)
    wd = _dot(sstack, wide)
    x1s = _to_stacked(wd[:N, :B * D], D).astype(BF16)
    x2s = _to_stacked(wd[N:, :B * D], D).astype(BF16)
    h1s_l0 = _to_stacked(wd[:N, B * D:2 * B * D], H).astype(BF16)
    h2s_l0 = _to_stacked(wd[N:, B * D:2 * B * D], H).astype(BF16)
    h1s_l1 = _to_stacked(wd[:N, 2 * B * D:], H).astype(BF16)
    h2s_l1 = _to_stacked(wd[N:, 2 * B * D:], H).astype(BF16)

    new0 = _layer(x0s, x1s, x2s, h0s_l0, h0bs_l0, h1s_l0, h2s_l0,
                  sstack, w0_ref[...], b0_ref[0], wg0_ref[...], wc0_ref[...])
    st0_ref[...] = new0

    y0s = new0.astype(BF16)
    xd = _dot(sstack, _to_lane(y0s))
    y1s = _to_stacked(xd[:N], H).astype(BF16)
    y2s = _to_stacked(xd[N:], H).astype(BF16)
    new1 = _layer(y0s, y1s, y2s, h0s_l1, h0bs_l1, h1s_l1, h2s_l1,
                  sstack, w1_ref[...], b1_ref[0], wg1_ref[...], wc1_ref[...])
    st1_ref[...] = new1

    for bi in range(B):
        @pl.when(t == idx_ref[bi])
        def _(bi=bi):
            last_ref[bi * N:(bi + 1) * N] = new1[bi * N:(bi + 1) * N]

    @pl.when(t == T - 1)
    def _():
        wfc = wfc_ref[...]
        bfc = bfc_ref[0]
        lg = _dot(jnp.maximum(last_ref[...], 0.0), wfc) + bfc
        o_ref[...] = jnp.max(lg.reshape(B, N, 128), axis=1)


def _mega(idx, x, s, w0_in, bias0, wg0_h, wc0_h, w1_in, bias1, wg1_h, wc1_h,
          wfc_pad, bfc_pad):
    return pl.pallas_call(
        _mega_body,
        grid=(T,),
        in_specs=[
            pl.BlockSpec(memory_space=pltpu.SMEM),
            pl.BlockSpec((B, 1, N, D), lambda t: (0, t, 0, 0)),
            pl.BlockSpec((N, N), lambda t: (0, 0)),
            pl.BlockSpec((K * D, 3 * H), lambda t: (0, 0)),
            pl.BlockSpec((1, 3 * H), lambda t: (0, 0)),
            pl.BlockSpec((K * H, 2 * H), lambda t: (0, 0)),
            pl.BlockSpec((K * H, H), lambda t: (0, 0)),
            pl.BlockSpec((K * H, 3 * H), lambda t: (0, 0)),
            pl.BlockSpec((1, 3 * H), lambda t: (0, 0)),
            pl.BlockSpec((K * H, 2 * H), lambda t: (0, 0)),
            pl.BlockSpec((K * H, H), lambda t: (0, 0)),
            pl.BlockSpec((H, 128), lambda t: (0, 0)),
            pl.BlockSpec((1, 128), lambda t: (0, 0)),
        ],
        out_specs=pl.BlockSpec((B, 128), lambda t: (0, 0)),
        out_shape=jax.ShapeDtypeStruct((B, 128), F32),
        scratch_shapes=[
            pltpu.VMEM((2 * N, N), BF16),
            pltpu.VMEM((B * N, H), F32),
            pltpu.VMEM((B * N, H), F32),
            pltpu.VMEM((B * N, H), F32),
        ],
    )(idx, x, s, w0_in, bias0, wg0_h, wc0_h, w1_in, bias1, wg1_h, wc1_h,
      wfc_pad, bfc_pad)


def _split_weight(w, din, dout):
    wr = w.reshape(din + H, K, dout)
    w_in = wr[:din].transpose(1, 0, 2).reshape(K * din, dout)
    w_h = wr[din:].transpose(1, 0, 2).reshape(K * H, dout)
    return w_in, w_h


def kernel(input_seq, seq_lengths, supports, Wg0, bg0, Wc0, bc0,
           Wg1, bg1, Wc1, bc1, Wfc, bfc):
    s = supports[0].astype(BF16)

    wg0_in, wg0_h = _split_weight(Wg0, D, 2 * H)
    wc0_in, wc0_h = _split_weight(Wc0, D, H)
    wg1_in, wg1_h = _split_weight(Wg1, H, 2 * H)
    wc1_in, wc1_h = _split_weight(Wc1, H, H)
    w0_in = jnp.concatenate([wg0_in, wc0_in], axis=1).astype(BF16)
    w1_in = jnp.concatenate([wg1_in, wc1_in], axis=1).astype(BF16)
    wg0_h = wg0_h.astype(BF16)
    wc0_h = wc0_h.astype(BF16)
    wg1_h = wg1_h.astype(BF16)
    wc1_h = wc1_h.astype(BF16)
    bias0 = jnp.concatenate([bg0, bc0]).reshape(1, 3 * H)
    bias1 = jnp.concatenate([bg1, bc1]).reshape(1, 3 * H)

    idx = jnp.clip(seq_lengths - 1, 0, T - 1).astype(jnp.int32)

    wfc_pad = jnp.zeros((H, 128), BF16).at[:, :C].set(Wfc.astype(BF16))
    bfc_pad = jnp.zeros((1, 128), F32).at[0, :C].set(bfc)

    logits_pad = _mega(idx, input_seq, s, w0_in, bias0, wg0_h, wc0_h,
                       w1_in, bias1, wg1_h, wc1_h, wfc_pad, bfc_pad)
    return logits_pad[:, :C]
